# Initial kernel scaffold; baseline (speedup 1.0000x reference)
import jax
import jax.numpy as jnp
from jax.experimental import pallas as pl


def kernel(points, index_grid, fc_pos_W, fc_pos_b, blocks_W0, blocks_b0, blocks_W1, blocks_b1, blocks_Ws, fc_c_W, fc_c_b):
    raise NotImplementedError("write your pallas kernel here")


# speedup vs baseline: 163.6571x; 163.6571x over previous
import functools

import jax
import jax.numpy as jnp
from jax import lax
from jax.experimental import pallas as pl
from jax.experimental.pallas import tpu as pltpu
from jax.experimental.pallas import tpu_sc as plsc

_B, _N, _D = 4, 16384, 3
_H = 128
_C = 128
_RESO = 32
_G = 2 * _RESO ** 3
_NB = _N // 16
_GB = _G // 16
_NBLK = 2048
_HB = 2
_NC = (_HB * _H) // 32


def _sc_mesh():
    return plsc.VectorSubcoreMesh(core_axis_name="c", subcore_axis_name="s")


def _wid():
    return lax.axis_index("s") * 2 + lax.axis_index("c")


def _tile_cols():
    wid = _wid()
    return wid // 16, (wid % 16) * _NC


def _pool_cols(idx_v, cols, tab_v, x_hbm, out_hbm, b, f0, sem_in, sem_out):
    zeros = jnp.zeros((16,), jnp.float32)
    in_h = [None] * _NC
    out_h = [None] * _NC
    in_h[0] = pltpu.async_copy(x_hbm.at[b, f0], cols[0], sem_in)
    for k in range(_NC):
        buf = cols[k % 2]
        if k >= 1:
            out_h[k - 1].wait()
        if k < _NC - 1:
            in_h[k + 1] = pltpu.async_copy(
                x_hbm.at[b, f0 + k + 1], cols[(k + 1) % 2], sem_in)
        in_h[k].wait()

        @plsc.parallel_loop(0, _NB, unroll=8)
        def _zb(i):
            ids = idx_v[pl.ds(i * 16, 16)]
            plsc.store_scatter(tab_v, [ids], zeros)

        @plsc.parallel_loop(0, _NB, unroll=8)
        def _ab(i):
            ids = idx_v[pl.ds(i * 16, 16)]
            vals = buf[pl.ds(i * 16, 16)]
            plsc.addupdate_scatter(tab_v, [ids], vals)

        @plsc.parallel_loop(0, _NB, unroll=8)
        def _gb(i):
            ids = idx_v[pl.ds(i * 16, 16)]
            buf[pl.ds(i * 16, 16)] = plsc.load_gather(tab_v, [ids])

        out_h[k] = pltpu.async_copy(buf, out_hbm.at[b, f0 + k], sem_out)
    out_h[_NC - 1].wait()


def _pool_body(idx_hbm, x_hbm, out_hbm, idx_v, colA, colB, tab_v, sem_in,
               sem_out):
    b, f0 = _tile_cols()
    pltpu.sync_copy(idx_hbm.at[b], idx_v)
    _pool_cols(idx_v, [colA, colB], tab_v, x_hbm, out_hbm, b, f0, sem_in,
               sem_out)


def _pool_rpt_body(idx_hbm, x_hbm, out_hbm, rpt_hbm, idx_v, colA, colB,
                   tab_v, sem_in, sem_out):
    wid = _wid()
    b, f0 = _tile_cols()
    zeros = jnp.zeros((16,), jnp.float32)
    ones = jnp.full((16,), 1.0, jnp.float32)
    pltpu.sync_copy(idx_hbm.at[b], idx_v)
    _pool_cols(idx_v, [colA, colB], tab_v, x_hbm, out_hbm, b, f0, sem_in,
               sem_out)

    @pl.when(wid % 16 == 0)
    def _():
        @plsc.parallel_loop(0, _NB, unroll=8)
        def _zb(i):
            ids = idx_v[pl.ds(i * 16, 16)]
            plsc.store_scatter(tab_v, [ids], zeros)

        @plsc.parallel_loop(0, _NB, unroll=8)
        def _ab(i):
            ids = idx_v[pl.ds(i * 16, 16)]
            plsc.addupdate_scatter(tab_v, [ids], ones)

        @plsc.parallel_loop(0, _NB, unroll=8)
        def _gb(i):
            ids = idx_v[pl.ds(i * 16, 16)]
            g = plsc.load_gather(tab_v, [ids])
            colA[pl.ds(i * 16, 16)] = 1.0 / g

        pltpu.sync_copy(colA, rpt_hbm.at[b])


def _final_body(idx_hbm, c_hbm, rpt_hbm, out_hbm, idx_v, colA, colB, rpt_v,
                tab_v, sem_in):
    b, f0 = _tile_cols()
    zeros = jnp.zeros((16,), jnp.float32)

    pltpu.sync_copy(idx_hbm.at[b], idx_v)
    pltpu.sync_copy(rpt_hbm.at[b], rpt_v)

    @plsc.parallel_loop(0, _GB, unroll=16)
    def _z0(i):
        tab_v[pl.ds(i * 16, 16)] = zeros

    cols = [colA, colB]
    in_h = [None] * _NC
    in_h[0] = pltpu.async_copy(c_hbm.at[b, f0], cols[0], sem_in)
    for k in range(_NC):
        f = f0 + k
        buf = cols[k % 2]
        if k < _NC - 1:
            in_h[k + 1] = pltpu.async_copy(
                c_hbm.at[b, f + 1], cols[(k + 1) % 2], sem_in)
        in_h[k].wait()

        @plsc.parallel_loop(0, _NB, unroll=8)
        def _ab(i):
            ids = idx_v[pl.ds(i * 16, 16)]
            vals = buf[pl.ds(i * 16, 16)] * rpt_v[pl.ds(i * 16, 16)]
            plsc.addupdate_scatter(tab_v, [ids], vals)

        pltpu.sync_copy(tab_v.at[pl.ds(0, _G // 2)], out_hbm.at[b, 2 * f])
        pltpu.sync_copy(tab_v.at[pl.ds(_G // 2, _G // 2)],
                        out_hbm.at[b, 2 * f + 1])

        if k < _NC - 1:
            @plsc.parallel_loop(0, _NB, unroll=8)
            def _zb(i):
                ids = idx_v[pl.ds(i * 16, 16)]
                plsc.store_scatter(tab_v, [ids], zeros)


def _sc_scratch(extra_col=0, extra_sem=0):
    return ([pltpu.VMEM((_N,), jnp.int32)]
            + [pltpu.VMEM((_N,), jnp.float32)] * (2 + extra_col)
            + [pltpu.VMEM((_G,), jnp.float32)]
            + [pltpu.SemaphoreType.DMA] * (1 + extra_sem))


def _run_pool(idx, x):
    run = functools.partial(
        pl.kernel,
        out_type=jax.ShapeDtypeStruct((_HB, _H, _N), jnp.float32),
        mesh=_sc_mesh(),
        compiler_params=pltpu.CompilerParams(needs_layout_passes=False),
        scratch_types=_sc_scratch(extra_sem=1),
    )(_pool_body)
    return run(idx, x)


def _run_pool_rpt(idx, x):
    run = functools.partial(
        pl.kernel,
        out_type=(jax.ShapeDtypeStruct((_HB, _H, _N), jnp.float32),
                  jax.ShapeDtypeStruct((_HB, _N), jnp.float32)),
        mesh=_sc_mesh(),
        compiler_params=pltpu.CompilerParams(needs_layout_passes=False),
        scratch_types=_sc_scratch(extra_sem=1),
    )(_pool_rpt_body)
    return run(idx, x)


def _run_final(idx, c, rpt):
    run = functools.partial(
        pl.kernel,
        out_type=jax.ShapeDtypeStruct((_HB, 2 * _C, _G // 2), jnp.float32),
        mesh=_sc_mesh(),
        compiler_params=pltpu.CompilerParams(needs_layout_passes=False),
        scratch_types=_sc_scratch(extra_col=1),
    )(_final_body)
    return run(idx, c, rpt)


def _dot(a, b):
    return jnp.dot(a, b, preferred_element_type=jnp.float32)


def _head_tc(pts_ref, wp, bp, w0, b0, w1, b1, ws, o_ref):
    pts = pts_ref[0]
    x = _dot(wp[...], pts) + bp[...]
    h = _dot(w0[...], jnp.maximum(x, 0.0)) + b0[...]
    dx = _dot(w1[...], jnp.maximum(h, 0.0)) + b1[...]
    o_ref[0] = _dot(ws[...], x) + dx


def _block_tc(net_ref, praw_ref, rpt_ref, w0a, w0b, b0, w1, b1, wsa, wsb,
              o_ref):
    xa = net_ref[0]
    p = praw_ref[0] * rpt_ref[0]
    h = (_dot(w0a[...], jnp.maximum(xa, 0.0))
         + _dot(w0b[...], jnp.maximum(p, 0.0)) + b0[...])
    dx = _dot(w1[...], jnp.maximum(h, 0.0)) + b1[...]
    o_ref[0] = _dot(wsa[...], xa) + _dot(wsb[...], p) + dx


def _blockc_tc(net_ref, praw_ref, rpt_ref, w0a, w0b, b0, w1, b1, wsa, wsb,
               wc, bc, o_ref):
    xa = net_ref[0]
    p = praw_ref[0] * rpt_ref[0]
    h = (_dot(w0a[...], jnp.maximum(xa, 0.0))
         + _dot(w0b[...], jnp.maximum(p, 0.0)) + b0[...])
    dx = _dot(w1[...], jnp.maximum(h, 0.0)) + b1[...]
    o = _dot(wsa[...], xa) + _dot(wsb[...], p) + dx
    o_ref[0] = _dot(wc[...], o) + bc[...]


def _fullspec(shp):
    return pl.BlockSpec(shp, lambda b, j: (0,) * len(shp))


def _run_head(pts8, wpT, bp, w0T, b0, w1T, b1, wsT):
    return pl.pallas_call(
        _head_tc,
        grid=(_HB, _N // _NBLK),
        in_specs=[
            pl.BlockSpec((1, 8, _NBLK), lambda b, j: (b, 0, j)),
            _fullspec((2 * _H, 8)), _fullspec((2 * _H, 1)),
            _fullspec((_H, 2 * _H)), _fullspec((_H, 1)),
            _fullspec((_H, _H)), _fullspec((_H, 1)),
            _fullspec((_H, 2 * _H)),
        ],
        out_specs=pl.BlockSpec((1, _H, _NBLK), lambda b, j: (b, 0, j)),
        out_shape=jax.ShapeDtypeStruct((_HB, _H, _N), jnp.float32),
    )(pts8, wpT, bp, w0T, b0, w1T, b1, wsT)


def _run_block(net, praw, rpt3, w0a, w0b, b0, w1, b1, wsa, wsb):
    return pl.pallas_call(
        _block_tc,
        grid=(_HB, _N // _NBLK),
        in_specs=[
            pl.BlockSpec((1, _H, _NBLK), lambda b, j: (b, 0, j)),
            pl.BlockSpec((1, _H, _NBLK), lambda b, j: (b, 0, j)),
            pl.BlockSpec((1, 1, _NBLK), lambda b, j: (b, 0, j)),
            _fullspec((_H, _H)), _fullspec((_H, _H)), _fullspec((_H, 1)),
            _fullspec((_H, _H)), _fullspec((_H, 1)),
            _fullspec((_H, _H)), _fullspec((_H, _H)),
        ],
        out_specs=pl.BlockSpec((1, _H, _NBLK), lambda b, j: (b, 0, j)),
        out_shape=jax.ShapeDtypeStruct((_HB, _H, _N), jnp.float32),
    )(net, praw, rpt3, w0a, w0b, b0, w1, b1, wsa, wsb)


def _run_blockc(net, praw, rpt3, w0a, w0b, b0, w1, b1, wsa, wsb, wc, bc):
    return pl.pallas_call(
        _blockc_tc,
        grid=(_HB, _N // _NBLK),
        in_specs=[
            pl.BlockSpec((1, _H, _NBLK), lambda b, j: (b, 0, j)),
            pl.BlockSpec((1, _H, _NBLK), lambda b, j: (b, 0, j)),
            pl.BlockSpec((1, 1, _NBLK), lambda b, j: (b, 0, j)),
            _fullspec((_H, _H)), _fullspec((_H, _H)), _fullspec((_H, 1)),
            _fullspec((_H, _H)), _fullspec((_H, 1)),
            _fullspec((_H, _H)), _fullspec((_H, _H)),
            _fullspec((_C, _H)), _fullspec((_C, 1)),
        ],
        out_specs=pl.BlockSpec((1, _C, _NBLK), lambda b, j: (b, 0, j)),
        out_shape=jax.ShapeDtypeStruct((_HB, _C, _N), jnp.float32),
    )(net, praw, rpt3, w0a, w0b, b0, w1, b1, wsa, wsb, wc, bc)


def kernel(points, index_grid, fc_pos_W, fc_pos_b, blocks_W0, blocks_b0,
           blocks_W1, blocks_b1, blocks_Ws, fc_c_W, fc_c_b):
    f32 = jnp.float32
    pts_t = jnp.transpose(points, (0, 2, 1))
    pts8 = jnp.concatenate([pts_t, jnp.zeros((_B, 8 - _D, _N), f32)], axis=1)
    wpT = jnp.concatenate(
        [fc_pos_W.T, jnp.zeros((2 * _H, 8 - _D), f32)], axis=1)
    bp = fc_pos_b.reshape(2 * _H, 1)
    W0T = jnp.transpose(blocks_W0, (0, 2, 1))
    W1T = jnp.transpose(blocks_W1, (0, 2, 1))
    WsT = jnp.transpose(blocks_Ws, (0, 2, 1))
    b0c = blocks_b0[..., None]
    b1c = blocks_b1[..., None]
    wcT = fc_c_W.T
    bc = fc_c_b.reshape(_C, 1)

    idx = index_grid.astype(jnp.int32)
    idx_h = [idx[0:_HB], idx[_HB:]]

    nets = [
        _run_head(pts8[h * _HB:(h + 1) * _HB], wpT, bp, W0T[0], b0c[0],
                  W1T[0], b1c[0], WsT[0])
        for h in range(2)
    ]
    rpts = [None, None]
    rpt3s = [None, None]

    for i in range(1, 5):
        praws = [None, None]
        for h in range(2):
            if i == 1:
                praws[h], rpts[h] = _run_pool_rpt(idx_h[h], nets[h])
                rpt3s[h] = rpts[h].reshape(_HB, 1, _N)
            else:
                praws[h] = _run_pool(idx_h[h], nets[h])
        w0a, w0b = W0T[i][:, :_H], W0T[i][:, _H:]
        wsa, wsb = WsT[i][:, :_H], WsT[i][:, _H:]
        for h in range(2):
            if i < 4:
                nets[h] = _run_block(nets[h], praws[h], rpt3s[h], w0a, w0b,
                                     b0c[i], W1T[i], b1c[i], wsa, wsb)
            else:
                nets[h] = _run_blockc(nets[h], praws[h], rpt3s[h], w0a, w0b,
                                      b0c[i], W1T[i], b1c[i], wsa, wsb,
                                      wcT, bc)

    outs = [_run_final(idx_h[h], nets[h], rpts[h]) for h in range(2)]
    out = jnp.concatenate(outs, axis=0)
    return out.reshape(_B, 2 * _C, _RESO, _RESO, _RESO)
